# Initial kernel scaffold; baseline (speedup 1.0000x reference)
import jax
import jax.numpy as jnp
from jax.experimental import pallas as pl


def kernel(x, edge_index, W1, b1, W2, b2, W3, b3):
    raise NotImplementedError("write your pallas kernel here")


# speedup vs baseline: 3.0854x; 3.0854x over previous
import functools

import jax
import jax.numpy as jnp
from jax import lax
from jax.experimental import pallas as pl
from jax.experimental.pallas import tpu as pltpu
from jax.experimental.pallas import tpu_sc as plsc

_N = 10000
_E = 320000
_D = 128
_NC = 2
_NS = 16
_NW = _NC * _NS
_NP = 10240
_RPT = _NP // _NS
_CH = 80
_EPT = _CH * 128
_EP = _NW * _EPT
_TB = 512

_mesh = plsc.VectorSubcoreMesh(core_axis_name="c", subcore_axis_name="s")


@functools.partial(
    pl.kernel,
    out_type=jax.ShapeDtypeStruct((_NC, 2, _NP), jnp.float32),
    mesh=_mesh,
    scratch_types=[
        pltpu.VMEM_SHARED((_NP,), jnp.float32),
        pltpu.VMEM_SHARED((_NP,), jnp.float32),
        pltpu.VMEM((_CH, 2, 128), jnp.int32),
        pltpu.VMEM((_RPT,), jnp.float32),
        pltpu.VMEM((128,), jnp.float32),
        pltpu.SemaphoreType.DMA,
    ],
)
def _deg_kernel(edge_hbm, out_hbm, acc_s, acc_d, pair_v, zb, ones_v, sem):
    cid = lax.axis_index("c")
    sid = lax.axis_index("s")
    wid = cid * _NS + sid
    pltpu.sync_copy(edge_hbm.at[wid], pair_v)

    def zfill(k, carry):
        zb[pl.ds(k * 16, 16)] = jnp.zeros((16,), jnp.float32)
        return carry

    lax.fori_loop(0, _RPT // 16, zfill, 0)

    def ofill(k, carry):
        ones_v[pl.ds(k * 16, 16)] = jnp.ones((16,), jnp.float32)
        return carry

    lax.fori_loop(0, 8, ofill, 0)

    base = sid * _RPT
    pltpu.sync_copy(zb, acc_s.at[pl.ds(base, _RPT)])
    pltpu.sync_copy(zb, acc_d.at[pl.ds(base, _RPT)])
    plsc.subcore_barrier()

    def fire(j, carry):
        pltpu.async_copy(ones_v, acc_s.at[pair_v.at[j, 0]], sem, add=True)
        pltpu.async_copy(ones_v, acc_d.at[pair_v.at[j, 1]], sem, add=True)
        return carry

    lax.fori_loop(0, _CH, fire, 0)

    def drain(j, carry):
        pltpu.make_async_copy(ones_v, acc_s.at[pair_v.at[j, 0]], sem).wait()
        pltpu.make_async_copy(ones_v, acc_d.at[pair_v.at[j, 1]], sem).wait()
        return carry

    lax.fori_loop(0, _CH, drain, 0)
    plsc.subcore_barrier()
    pltpu.sync_copy(acc_s.at[pl.ds(base, _RPT)], out_hbm.at[cid, 0, pl.ds(base, _RPT)])
    pltpu.sync_copy(acc_d.at[pl.ds(base, _RPT)], out_hbm.at[cid, 1, pl.ds(base, _RPT)])


@functools.partial(
    pl.kernel,
    out_type=jax.ShapeDtypeStruct((_NC, _NP, _D), jnp.float32),
    mesh=_mesh,
    scratch_types=[
        pltpu.VMEM_SHARED((_NP, _D), jnp.float32),
        pltpu.VMEM((2, 128), jnp.int32),
        pltpu.VMEM((2, 128), jnp.int32),
        pltpu.VMEM((128, _D), jnp.float32),
        pltpu.VMEM((128, _D), jnp.float32),
        pltpu.SemaphoreType.DMA,
        pltpu.SemaphoreType.DMA,
        pltpu.SemaphoreType.DMA,
        pltpu.SemaphoreType.DMA,
    ],
)
def _agg_kernel(g_hbm, edge_hbm, out_hbm,
                acc, pair0, pair1, rows0, rows1, isem0, isem1, gsem0, gsem1):
    cid = lax.axis_index("c")
    sid = lax.axis_index("s")
    wid = cid * _NS + sid

    def zfill(k, carry):
        rows0[k // 8, pl.ds((k % 8) * 16, 16)] = jnp.zeros((16,), jnp.float32)
        return carry

    lax.fori_loop(0, 128 * 8, zfill, 0)

    base = sid * _RPT
    for t in range(_RPT // 128):
        pltpu.sync_copy(rows0, acc.at[pl.ds(base + t * 128, 128)])
    plsc.subcore_barrier()

    pairs = (pair0, pair1)
    rows = (rows0, rows1)
    isems = (isem0, isem1)
    gsems = (gsem0, gsem1)

    pltpu.sync_copy(edge_hbm.at[wid, 0], pair0)
    pltpu.async_copy(edge_hbm.at[wid, 1], pair1, isem1)
    pltpu.async_copy(g_hbm.at[pair0.at[0]], rows0, gsem0)

    def body(jj, carry):
        for p in range(2):
            j = jj * 2 + p
            q = (p + 1) % 2

            @pl.when(j + 1 < _CH)
            def _next_gather():
                pltpu.make_async_copy(edge_hbm.at[wid, j + 1], pairs[q],
                                      isems[q]).wait()
                pltpu.async_copy(g_hbm.at[pairs[q].at[0]], rows[q], gsems[q])

            pltpu.make_async_copy(g_hbm.at[pairs[p].at[0]], rows[p],
                                  gsems[p]).wait()
            pltpu.sync_copy(rows[p], acc.at[pairs[p].at[1]], add=True)

            @pl.when(j + 2 < _CH)
            def _next_pair():
                pltpu.async_copy(edge_hbm.at[wid, j + 2], pairs[p], isems[p])
        return carry

    lax.fori_loop(0, _CH // 2, body, 0)
    plsc.subcore_barrier()
    pltpu.sync_copy(acc.at[pl.ds(base, _RPT)], out_hbm.at[cid, pl.ds(base, _RPT)])


def _prescale_body(x_ref, deg_ref, o_ref):
    ds = deg_ref[0, 0] + deg_ref[1, 0]
    o_ref[...] = x_ref[...] * lax.rsqrt(jnp.maximum(ds, 1.0))


def _layer_body(a_ref, deg_ref, w_ref, b_ref, o_ref, *, relu, prescale):
    agg = a_ref[0] + a_ref[1]
    dd = deg_ref[0, 1] + deg_ref[1, 1]
    h = agg * lax.rsqrt(jnp.maximum(dd, 1.0))
    h = jnp.dot(h, w_ref[...], preferred_element_type=jnp.float32) + b_ref[...]
    if relu:
        h = jnp.maximum(h, 0.0)
    if prescale:
        ds = deg_ref[0, 0] + deg_ref[1, 0]
        h = h * lax.rsqrt(jnp.maximum(ds, 1.0))
    o_ref[...] = h


_deg_spec = pl.BlockSpec((2, 2, _TB, 1), lambda i: (0, 0, i, 0))

_prescale = pl.pallas_call(
    _prescale_body,
    grid=(_NP // _TB,),
    in_specs=[pl.BlockSpec((_TB, _D), lambda i: (i, 0)), _deg_spec],
    out_specs=pl.BlockSpec((_TB, _D), lambda i: (i, 0)),
    out_shape=jax.ShapeDtypeStruct((_NP, _D), jnp.float32),
)


def _make_layer(relu, prescale):
    return pl.pallas_call(
        functools.partial(_layer_body, relu=relu, prescale=prescale),
        grid=(_NP // _TB,),
        in_specs=[
            pl.BlockSpec((2, _TB, _D), lambda i: (0, i, 0)),
            _deg_spec,
            pl.BlockSpec((_D, _D), lambda i: (0, 0)),
            pl.BlockSpec((1, _D), lambda i: (0, 0)),
        ],
        out_specs=pl.BlockSpec((_TB, _D), lambda i: (i, 0)),
        out_shape=jax.ShapeDtypeStruct((_NP, _D), jnp.float32),
    )


_layer_mid = _make_layer(relu=True, prescale=True)
_layer_last = _make_layer(relu=False, prescale=False)


def kernel(x, edge_index, W1, b1, W2, b2, W3, b3):
    src = edge_index[0].astype(jnp.int32)
    dst = edge_index[1].astype(jnp.int32)
    pad = _EP - _E
    src = jnp.concatenate([src, jnp.full((pad,), _N, jnp.int32)])
    dst = jnp.concatenate([dst, jnp.full((pad,), _N, jnp.int32)])
    edges = jnp.stack([src.reshape(_NW, _CH, 128),
                       dst.reshape(_NW, _CH, 128)], axis=2)

    degp = _deg_kernel(edges)
    degr = degp.reshape(2, 2, _NP, 1)

    xp = jnp.pad(x, ((0, _NP - _N), (0, 0)))
    g = _prescale(xp, degr)
    h = _layer_mid(_agg_kernel(g, edges), degr, W1, b1.reshape(1, _D))
    h = _layer_mid(_agg_kernel(h, edges), degr, W2, b2.reshape(1, _D))
    out = _layer_last(_agg_kernel(h, edges), degr, W3, b3.reshape(1, _D))
    return out[:_N]


# speedup vs baseline: 4.0621x; 1.3165x over previous
import functools

import jax
import jax.numpy as jnp
from jax import lax
from jax.experimental import pallas as pl
from jax.experimental.pallas import tpu as pltpu
from jax.experimental.pallas import tpu_sc as plsc

_N = 10000
_E = 320000
_D = 128
_NC = 2
_NS = 16
_NW = _NC * _NS
_NP = 10240
_RPT = _NP // _NS
_CH = 80
_EPT = _CH * 128
_EP = _NW * _EPT
_TB = 512
_C0 = 152
_C1 = 8

_mesh = plsc.VectorSubcoreMesh(core_axis_name="c", subcore_axis_name="s")


@functools.partial(
    pl.kernel,
    out_type=jax.ShapeDtypeStruct((_NC, 2, _NP), jnp.float32),
    mesh=_mesh,
    scratch_types=[
        pltpu.VMEM_SHARED((_NP,), jnp.float32),
        pltpu.VMEM_SHARED((_NP,), jnp.float32),
        pltpu.VMEM((_CH, 2, 128), jnp.int32),
        pltpu.VMEM((_RPT,), jnp.float32),
        pltpu.VMEM((128,), jnp.float32),
        pltpu.SemaphoreType.DMA,
    ],
)
def _deg_kernel(edge_hbm, out_hbm, acc_s, acc_d, pair_v, zb, ones_v, sem):
    cid = lax.axis_index("c")
    sid = lax.axis_index("s")
    wid = cid * _NS + sid
    pltpu.sync_copy(edge_hbm.at[wid], pair_v)

    def zfill(k, carry):
        zb[pl.ds(k * 16, 16)] = jnp.zeros((16,), jnp.float32)
        return carry

    lax.fori_loop(0, _RPT // 16, zfill, 0)

    def ofill(k, carry):
        ones_v[pl.ds(k * 16, 16)] = jnp.ones((16,), jnp.float32)
        return carry

    lax.fori_loop(0, 8, ofill, 0)

    base = sid * _RPT
    pltpu.sync_copy(zb, acc_s.at[pl.ds(base, _RPT)])
    pltpu.sync_copy(zb, acc_d.at[pl.ds(base, _RPT)])
    plsc.subcore_barrier()

    def fire(j, carry):
        pltpu.async_copy(ones_v, acc_s.at[pair_v.at[j, 0]], sem, add=True)
        pltpu.async_copy(ones_v, acc_d.at[pair_v.at[j, 1]], sem, add=True)
        return carry

    lax.fori_loop(0, _CH, fire, 0)

    def drain(j, carry):
        pltpu.make_async_copy(ones_v, acc_s.at[pair_v.at[j, 0]], sem).wait()
        pltpu.make_async_copy(ones_v, acc_d.at[pair_v.at[j, 1]], sem).wait()
        return carry

    lax.fori_loop(0, _CH, drain, 0)
    plsc.subcore_barrier()
    pltpu.sync_copy(acc_s.at[pl.ds(base, _RPT)], out_hbm.at[cid, 0, pl.ds(base, _RPT)])
    pltpu.sync_copy(acc_d.at[pl.ds(base, _RPT)], out_hbm.at[cid, 1, pl.ds(base, _RPT)])


@functools.partial(
    pl.kernel,
    out_type=jax.ShapeDtypeStruct((_NC, _NP, _D), jnp.float32),
    mesh=_mesh,
    scratch_types=[
        pltpu.VMEM_SHARED((_NP, _D), jnp.float32),
        [pltpu.VMEM((2, 128), jnp.int32) for _ in range(8)],
        [pltpu.VMEM((128, _D), jnp.float32) for _ in range(2)],
        [pltpu.SemaphoreType.DMA for _ in range(8)],
        [pltpu.SemaphoreType.DMA for _ in range(2)],
        [pltpu.SemaphoreType.DMA for _ in range(2)],
    ],
)
def _agg_kernel(g_hbm, edge_hbm, out_hbm, acc, pairs, rows, isems, gsems, ssems):
    cid = lax.axis_index("c")
    sid = lax.axis_index("s")
    cnt = jnp.where(cid == 0, _C0, _C1)
    start = jnp.where(cid == 0, sid * _C0, _NS * _C0 + sid * _C1)

    def zfill(k, carry):
        rows[0][k // 8, pl.ds((k % 8) * 16, 16)] = jnp.zeros((16,), jnp.float32)
        return carry

    lax.fori_loop(0, 128 * 8, zfill, 0)

    base = sid * _RPT
    for t in range(_RPT // 128):
        pltpu.sync_copy(rows[0], acc.at[pl.ds(base + t * 128, 128)])
    plsc.subcore_barrier()

    pltpu.sync_copy(edge_hbm.at[start], pairs[0])
    for k in range(1, 6):
        pltpu.async_copy(edge_hbm.at[start + k], pairs[k], isems[k])
    pltpu.async_copy(g_hbm.at[pairs[0].at[0]], rows[0], gsems[0])

    def body(jj, carry):
        for u in range(8):
            i = jj * 8 + u
            p = u % 2
            q = (u + 1) % 2
            s1 = (u + 1) % 8
            s6 = (u + 6) % 8

            @pl.when(i + 1 < cnt)
            def _next_gather():
                pltpu.make_async_copy(edge_hbm.at[start + i + 1], pairs[s1],
                                      isems[s1]).wait()

                @pl.when(i >= 1)
                def _rows_free():
                    pltpu.make_async_copy(rows[q], acc.at[pairs[s1].at[1]],
                                          ssems[q]).wait()

                pltpu.async_copy(g_hbm.at[pairs[s1].at[0]], rows[q], gsems[q])

            pltpu.make_async_copy(g_hbm.at[pairs[u].at[0]], rows[p],
                                  gsems[p]).wait()
            pltpu.async_copy(rows[p], acc.at[pairs[u].at[1]], ssems[p],
                             add=True)

            @pl.when(i + 6 < cnt)
            def _next_pair():
                pltpu.async_copy(edge_hbm.at[start + i + 6], pairs[s6],
                                 isems[s6])
        return carry

    lax.fori_loop(0, cnt // 8, body, 0)
    pltpu.make_async_copy(rows[0], acc.at[pairs[0].at[1]], ssems[0]).wait()
    pltpu.make_async_copy(rows[1], acc.at[pairs[1].at[1]], ssems[1]).wait()
    plsc.subcore_barrier()
    pltpu.sync_copy(acc.at[pl.ds(base, _RPT)], out_hbm.at[cid, pl.ds(base, _RPT)])


def _prescale_body(x_ref, deg_ref, o_ref):
    ds = deg_ref[0, 0] + deg_ref[1, 0]
    o_ref[...] = x_ref[...] * lax.rsqrt(jnp.maximum(ds, 1.0))


def _layer_body(a_ref, deg_ref, w_ref, b_ref, o_ref, *, relu, prescale):
    agg = a_ref[0] + a_ref[1]
    dd = deg_ref[0, 1] + deg_ref[1, 1]
    h = agg * lax.rsqrt(jnp.maximum(dd, 1.0))
    h = jnp.dot(h, w_ref[...], preferred_element_type=jnp.float32) + b_ref[...]
    if relu:
        h = jnp.maximum(h, 0.0)
    if prescale:
        ds = deg_ref[0, 0] + deg_ref[1, 0]
        h = h * lax.rsqrt(jnp.maximum(ds, 1.0))
    o_ref[...] = h


_deg_spec = pl.BlockSpec((2, 2, _TB, 1), lambda i: (0, 0, i, 0))

_prescale = pl.pallas_call(
    _prescale_body,
    grid=(_NP // _TB,),
    in_specs=[pl.BlockSpec((_TB, _D), lambda i: (i, 0)), _deg_spec],
    out_specs=pl.BlockSpec((_TB, _D), lambda i: (i, 0)),
    out_shape=jax.ShapeDtypeStruct((_NP, _D), jnp.float32),
)


def _make_layer(relu, prescale):
    return pl.pallas_call(
        functools.partial(_layer_body, relu=relu, prescale=prescale),
        grid=(_NP // _TB,),
        in_specs=[
            pl.BlockSpec((2, _TB, _D), lambda i: (0, i, 0)),
            _deg_spec,
            pl.BlockSpec((_D, _D), lambda i: (0, 0)),
            pl.BlockSpec((1, _D), lambda i: (0, 0)),
        ],
        out_specs=pl.BlockSpec((_TB, _D), lambda i: (i, 0)),
        out_shape=jax.ShapeDtypeStruct((_NP, _D), jnp.float32),
    )


_layer_mid = _make_layer(relu=True, prescale=True)
_layer_last = _make_layer(relu=False, prescale=False)


def kernel(x, edge_index, W1, b1, W2, b2, W3, b3):
    src = edge_index[0].astype(jnp.int32)
    dst = edge_index[1].astype(jnp.int32)
    pad = _EP - _E
    src = jnp.concatenate([src, jnp.full((pad,), _N, jnp.int32)])
    dst = jnp.concatenate([dst, jnp.full((pad,), _N, jnp.int32)])
    edges = jnp.stack([src.reshape(_NW, _CH, 128),
                       dst.reshape(_NW, _CH, 128)], axis=2)
    edges_f = edges.reshape(_NW * _CH, 2, 128)

    degp = _deg_kernel(edges)
    degr = degp.reshape(2, 2, _NP, 1)

    xp = jnp.pad(x, ((0, _NP - _N), (0, 0)))
    g = _prescale(xp, degr)
    h = _layer_mid(_agg_kernel(g, edges_f), degr, W1, b1.reshape(1, _D))
    h = _layer_mid(_agg_kernel(h, edges_f), degr, W2, b2.reshape(1, _D))
    out = _layer_last(_agg_kernel(h, edges_f), degr, W3, b3.reshape(1, _D))
    return out[:_N]
